# baseline (speedup 1.0000x reference)
import functools

import jax
import jax.numpy as jnp
from jax import lax
from jax.experimental import pallas as pl
from jax.experimental.pallas import tpu as pltpu
from jax.experimental.pallas import tpu_sc as plsc

D = 32
B = 4096
K = 20
NC = 2
NS = 16
NW = NC * NS
E_W = B // NW


def _sc_gather_body(table, pos_src, pos_dst, neg_src, neg_dst,
                    o_ps, o_pd, o_ns, o_nd,
                    idx_v, rows_v, acc_v, sem):
    wid = lax.axis_index("s") * NC + lax.axis_index("c")
    base = wid * E_W

    def pos_side(idx_hbm, out_hbm):
        pltpu.sync_copy(idx_hbm.at[pl.ds(base, E_W)], idx_v.at[0])
        pltpu.async_copy(table.at[idx_v.at[0]],
                         rows_v.at[pl.ds(0, E_W)], sem).wait()
        pltpu.sync_copy(rows_v.at[pl.ds(0, E_W)],
                        out_hbm.at[pl.ds(base, E_W)])

    pos_side(pos_src, o_ps)
    pos_side(pos_dst, o_pd)

    def neg_side(idx_hbm, out_hbm):
        pltpu.sync_copy(idx_hbm.at[wid], idx_v)
        copies = [
            pltpu.async_copy(table.at[idx_v.at[c]],
                             rows_v.at[pl.ds(c * E_W, E_W)], sem)
            for c in range(K)
        ]
        for cp in copies:
            cp.wait()

        def acc_edge(e, carry):
            r = e * K
            a0 = rows_v[r, pl.ds(0, 16)]
            a1 = rows_v[r, pl.ds(16, 16)]
            for kk in range(1, K):
                a0 = a0 + rows_v[r + kk, pl.ds(0, 16)]
                a1 = a1 + rows_v[r + kk, pl.ds(16, 16)]
            acc_v[e, pl.ds(0, 16)] = a0
            acc_v[e, pl.ds(16, 16)] = a1
            return carry

        lax.fori_loop(0, E_W, acc_edge, 0)
        pltpu.sync_copy(acc_v, out_hbm.at[pl.ds(base, E_W)])

    neg_side(neg_src, o_ns)
    neg_side(neg_dst, o_nd)


_sc_gather = functools.partial(
    pl.kernel,
    mesh=plsc.VectorSubcoreMesh(core_axis_name="c", subcore_axis_name="s"),
    out_type=[jax.ShapeDtypeStruct((B, D), jnp.float32)] * 4,
    scratch_types=[
        pltpu.VMEM((K, E_W), jnp.int32),
        pltpu.VMEM((K * E_W, D), jnp.float32),
        pltpu.VMEM((E_W, D), jnp.float32),
        pltpu.SemaphoreType.DMA,
    ],
    compiler_params=pltpu.CompilerParams(use_tc_tiling_on_sc=False),
)(_sc_gather_body)


def _tc_loss_body(ps_ref, pd_ref, ns_ref, nd_ref, w_ref, out_ref):
    eps = 1e-8
    epsilon = 1e-7
    w = w_ref[...]

    def norm(x):
        rms = jnp.sqrt(jnp.sum(x * x, axis=-1, keepdims=True) * (1.0 / D))
        return x / (rms + eps) * w

    a = norm(ps_ref[...])
    b = norm(pd_ref[...])
    c = norm(ns_ref[...] * (1.0 / K))
    d = norm(nd_ref[...] * (1.0 / K))
    dpos = jnp.sum(a * b, axis=-1, keepdims=True)
    dneg = jnp.sum(c * d, axis=-1, keepdims=True)

    def log_sig(x):
        return jnp.log(1.0 / (1.0 + jnp.exp(-x)) + epsilon)

    def log_one_minus_sig(x):
        return jnp.log(1.0 - 1.0 / (1.0 + jnp.exp(-x)) + epsilon)

    one = log_sig(dpos * (1.0 / D)) + log_one_minus_sig(dneg * (1.0 / D))
    two = log_sig(dpos) + log_one_minus_sig(dneg)
    out_ref[0, 0] = -(jnp.sum(one) + jnp.sum(two)) * (1.0 / B)


_tc_loss = pl.pallas_call(
    _tc_loss_body,
    out_shape=jax.ShapeDtypeStruct((1, 1), jnp.float32),
    out_specs=pl.BlockSpec(memory_space=pltpu.SMEM),
)


def kernel(pos_edges, neg_edges, node_embeddings, rms_weight):
    pos_src = pos_edges[0]
    pos_dst = pos_edges[1]
    neg_src = neg_edges[0].reshape(NW, K, E_W)
    neg_dst = neg_edges[1].reshape(NW, K, E_W)
    ps, pd, ns, nd = _sc_gather(node_embeddings, pos_src, pos_dst,
                                neg_src, neg_dst)
    loss = _tc_loss(ps, pd, ns, nd, rms_weight.reshape(1, D))
    return loss[0, 0]


# speedup vs baseline: 1.0002x; 1.0002x over previous
import functools

import jax
import jax.numpy as jnp
from jax import lax
from jax.experimental import pallas as pl
from jax.experimental.pallas import tpu as pltpu
from jax.experimental.pallas import tpu_sc as plsc

D = 32
B = 4096
K = 20
NUM_ROWS = 1000000
B_ROWS = NUM_ROWS // 4
NC = 2
NS = 16
NW = NC * NS
E_W = B // NW


def _sc_gather_body(table, pos_src, pos_dst, neg_src, neg_dst,
                    o_ps, o_pd, o_ns, o_nd,
                    idx_v, rows_v, acc_v, sem):
    wid = lax.axis_index("s") * NC + lax.axis_index("c")
    base = wid * E_W

    def pos_side(idx_hbm, out_hbm):
        pltpu.sync_copy(idx_hbm.at[pl.ds(base, E_W)], idx_v.at[0])
        pltpu.async_copy(table.at[idx_v.at[0]],
                         rows_v.at[pl.ds(0, E_W)], sem).wait()
        pltpu.sync_copy(rows_v.at[pl.ds(0, E_W)],
                        out_hbm.at[pl.ds(base, E_W)])

    pos_side(pos_src, o_ps)
    pos_side(pos_dst, o_pd)

    def neg_side(idx_hbm, out_hbm):
        pltpu.sync_copy(idx_hbm.at[wid], idx_v)
        copies = [
            pltpu.async_copy(table.at[idx_v.at[c]],
                             rows_v.at[pl.ds(c * E_W, E_W)], sem)
            for c in range(K)
        ]
        for cp in copies:
            cp.wait()

        def acc_edge(e, carry):
            r = e * K
            a0 = rows_v[r, pl.ds(0, 16)]
            a1 = rows_v[r, pl.ds(16, 16)]
            for kk in range(1, K):
                a0 = a0 + rows_v[r + kk, pl.ds(0, 16)]
                a1 = a1 + rows_v[r + kk, pl.ds(16, 16)]
            acc_v[e, pl.ds(0, 16)] = a0
            acc_v[e, pl.ds(16, 16)] = a1
            return carry

        lax.fori_loop(0, E_W, acc_edge, 0)
        pltpu.sync_copy(acc_v, out_hbm.at[pl.ds(base, E_W)])

    neg_side(neg_src, o_ns)
    neg_side(neg_dst, o_nd)


_sc_gather = functools.partial(
    pl.kernel,
    mesh=plsc.VectorSubcoreMesh(core_axis_name="c", subcore_axis_name="s"),
    out_type=[jax.ShapeDtypeStruct((B, D), jnp.float32)] * 4,
    scratch_types=[
        pltpu.VMEM((K, E_W), jnp.int32),
        pltpu.VMEM((K * E_W, D), jnp.float32),
        pltpu.VMEM((E_W, D), jnp.float32),
        pltpu.SemaphoreType.DMA,
    ],
    compiler_params=pltpu.CompilerParams(use_tc_tiling_on_sc=False),
)(_sc_gather_body)


def _tc_loss_body(ps_ref, pd_ref, ns_ref, nd_ref, w_ref, out_ref):
    eps = 1e-8
    epsilon = 1e-7
    w = w_ref[...]

    def norm(x):
        rms = jnp.sqrt(jnp.sum(x * x, axis=-1, keepdims=True) * (1.0 / D))
        return x / (rms + eps) * w

    a = norm(ps_ref[...])
    b = norm(pd_ref[...])
    c = norm(ns_ref[...] * (1.0 / K))
    d = norm(nd_ref[...] * (1.0 / K))
    dpos = jnp.sum(a * b, axis=-1, keepdims=True)
    dneg = jnp.sum(c * d, axis=-1, keepdims=True)

    def log_sig(x):
        return jnp.log(1.0 / (1.0 + jnp.exp(-x)) + epsilon)

    def log_one_minus_sig(x):
        return jnp.log(1.0 - 1.0 / (1.0 + jnp.exp(-x)) + epsilon)

    one = log_sig(dpos * (1.0 / D)) + log_one_minus_sig(dneg * (1.0 / D))
    two = log_sig(dpos) + log_one_minus_sig(dneg)
    out_ref[0, 0] = -(jnp.sum(one) + jnp.sum(two)) * (1.0 / B)


_tc_loss = pl.pallas_call(
    _tc_loss_body,
    out_shape=jax.ShapeDtypeStruct((1, 1), jnp.float32),
    out_specs=pl.BlockSpec(memory_space=pltpu.SMEM),
)


def kernel(pos_edges, neg_edges, node_embeddings, rms_weight):
    pos_src = pos_edges[0]
    pos_dst = pos_edges[1]
    neg_src = neg_edges[0].reshape(NW, K, E_W)
    neg_dst = neg_edges[1].reshape(NW, K, E_W)
    t128 = jax.lax.optimization_barrier(node_embeddings.reshape(B_ROWS, 128))
    table_lin = t128.reshape(NUM_ROWS, D)
    ps, pd, ns, nd = _sc_gather(table_lin, pos_src, pos_dst,
                                neg_src, neg_dst)
    loss = _tc_loss(ps, pd, ns, nd, rms_weight.reshape(1, D))
    return loss[0, 0]


# speedup vs baseline: 1.7403x; 1.7400x over previous
import functools

import jax
import jax.numpy as jnp
from jax import lax
from jax.experimental import pallas as pl
from jax.experimental.pallas import tpu as pltpu
from jax.experimental.pallas import tpu_sc as plsc

D = 32
B = 4096
K = 20
NUM_ROWS = 1000000
RC = 8192
RG = 123
PAD_ROWS = RG * RC
NC = 2
NS = 16
NW = NC * NS
E_W = B // NW


def _sc_gather_body(table, pos_src, pos_dst, neg_src, neg_dst,
                    o_ps, o_pd, o_ns, o_nd,
                    idx_v, rows_v, acc_v, sem):
    wid = lax.axis_index("s") * NC + lax.axis_index("c")
    base = wid * E_W

    def pos_side(idx_hbm, out_hbm):
        pltpu.sync_copy(idx_hbm.at[pl.ds(base, E_W)], idx_v.at[0])
        pltpu.async_copy(table.at[idx_v.at[0]],
                         rows_v.at[pl.ds(0, E_W)], sem).wait()
        pltpu.sync_copy(rows_v.at[pl.ds(0, E_W)],
                        out_hbm.at[pl.ds(base, E_W)])

    pos_side(pos_src, o_ps)
    pos_side(pos_dst, o_pd)

    def neg_side(idx_hbm, out_hbm):
        pltpu.sync_copy(idx_hbm.at[wid], idx_v)
        copies = [
            pltpu.async_copy(table.at[idx_v.at[c]],
                             rows_v.at[pl.ds(c * E_W, E_W)], sem)
            for c in range(K)
        ]
        for cp in copies:
            cp.wait()

        def acc_edge(e, carry):
            r = e * K
            a0 = rows_v[r, pl.ds(0, 16)]
            a1 = rows_v[r, pl.ds(16, 16)]
            for kk in range(1, K):
                a0 = a0 + rows_v[r + kk, pl.ds(0, 16)]
                a1 = a1 + rows_v[r + kk, pl.ds(16, 16)]
            acc_v[e, pl.ds(0, 16)] = a0
            acc_v[e, pl.ds(16, 16)] = a1
            return carry

        lax.fori_loop(0, E_W, acc_edge, 0)
        pltpu.sync_copy(acc_v, out_hbm.at[pl.ds(base, E_W)])

    neg_side(neg_src, o_ns)
    neg_side(neg_dst, o_nd)


_sc_gather = functools.partial(
    pl.kernel,
    mesh=plsc.VectorSubcoreMesh(core_axis_name="c", subcore_axis_name="s"),
    out_type=[jax.ShapeDtypeStruct((B, D), jnp.float32)] * 4,
    scratch_types=[
        pltpu.VMEM((K, E_W), jnp.int32),
        pltpu.VMEM((K * E_W, D), jnp.float32),
        pltpu.VMEM((E_W, D), jnp.float32),
        pltpu.SemaphoreType.DMA,
    ],
    compiler_params=pltpu.CompilerParams(use_tc_tiling_on_sc=False),
)(_sc_gather_body)


def _repack_body(at_ref, out_ref):
    yt = at_ref[...].T
    q = RC // 4
    out_ref[...] = jnp.concatenate(
        [yt[0:q], yt[q:2 * q], yt[2 * q:3 * q], yt[3 * q:4 * q]], axis=1)


_repack = pl.pallas_call(
    _repack_body,
    grid=(RG,),
    in_specs=[pl.BlockSpec((32, RC), lambda i: (0, i))],
    out_specs=pl.BlockSpec((RC // 4, 128), lambda i: (i, 0)),
    out_shape=jax.ShapeDtypeStruct((PAD_ROWS // 4, 128), jnp.float32),
)


def _permute_idx(r):
    return (r & ~(RC - 1)) + ((r & (RC // 4 - 1)) << 2) + ((r & (RC - 1)) >> 11)


def _tc_loss_body(ps_ref, pd_ref, ns_ref, nd_ref, w_ref, out_ref):
    eps = 1e-8
    epsilon = 1e-7
    w = w_ref[...]

    def norm(x):
        rms = jnp.sqrt(jnp.sum(x * x, axis=-1, keepdims=True) * (1.0 / D))
        return x / (rms + eps) * w

    a = norm(ps_ref[...])
    b = norm(pd_ref[...])
    c = norm(ns_ref[...] * (1.0 / K))
    d = norm(nd_ref[...] * (1.0 / K))
    dpos = jnp.sum(a * b, axis=-1, keepdims=True)
    dneg = jnp.sum(c * d, axis=-1, keepdims=True)

    def log_sig(x):
        return jnp.log(1.0 / (1.0 + jnp.exp(-x)) + epsilon)

    def log_one_minus_sig(x):
        return jnp.log(1.0 - 1.0 / (1.0 + jnp.exp(-x)) + epsilon)

    one = log_sig(dpos * (1.0 / D)) + log_one_minus_sig(dneg * (1.0 / D))
    two = log_sig(dpos) + log_one_minus_sig(dneg)
    out_ref[0, 0] = -(jnp.sum(one) + jnp.sum(two)) * (1.0 / B)


_tc_loss = pl.pallas_call(
    _tc_loss_body,
    out_shape=jax.ShapeDtypeStruct((1, 1), jnp.float32),
    out_specs=pl.BlockSpec(memory_space=pltpu.SMEM),
)


def kernel(pos_edges, neg_edges, node_embeddings, rms_weight):
    pos_src = _permute_idx(pos_edges[0])
    pos_dst = _permute_idx(pos_edges[1])
    neg_src = _permute_idx(neg_edges[0]).reshape(NW, K, E_W)
    neg_dst = _permute_idx(neg_edges[1]).reshape(NW, K, E_W)
    table_lin = _repack(node_embeddings.T).reshape(PAD_ROWS, D)
    ps, pd, ns, nd = _sc_gather(table_lin, pos_src, pos_dst,
                                neg_src, neg_dst)
    loss = _tc_loss(ps, pd, ns, nd, rms_weight.reshape(1, D))
    return loss[0, 0]


# speedup vs baseline: 2.6548x; 1.5255x over previous
import functools

import jax
import jax.numpy as jnp
from jax import lax
from jax.experimental import pallas as pl
from jax.experimental.pallas import tpu as pltpu
from jax.experimental.pallas import tpu_sc as plsc

D = 32
B = 4096
K = 20
NUM_ROWS = 1000000
RC = 8192
RG = 123
PAD_ROWS = RG * RC
NC = 2
NS = 16
NW = NC * NS
E_W = B // NW


def _sc_gather_body(table, pos_src, pos_dst, neg_src, neg_dst,
                    o_ps, o_pd, o_ns, o_nd,
                    idx_v, rows_v, acc_v, sem):
    wid = lax.axis_index("s") * NC + lax.axis_index("c")
    base = wid * E_W

    def pos_side(idx_hbm, out_hbm):
        pltpu.sync_copy(idx_hbm.at[pl.ds(base, E_W)], idx_v.at[0])
        pltpu.async_copy(table.at[idx_v.at[0]],
                         rows_v.at[pl.ds(0, E_W)], sem).wait()
        pltpu.sync_copy(rows_v.at[pl.ds(0, E_W)],
                        out_hbm.at[pl.ds(base, E_W)])

    pos_side(pos_src, o_ps)
    pos_side(pos_dst, o_pd)

    def neg_side(idx_hbm, out_hbm):
        pltpu.sync_copy(idx_hbm.at[wid], idx_v)
        copies = [
            pltpu.async_copy(table.at[idx_v.at[c]],
                             rows_v.at[pl.ds(c * E_W, E_W)], sem)
            for c in range(K)
        ]
        for cp in copies:
            cp.wait()

        def acc_edge(e, carry):
            r = e * K
            a0 = rows_v[r, pl.ds(0, 16)]
            a1 = rows_v[r, pl.ds(16, 16)]
            for kk in range(1, K):
                a0 = a0 + rows_v[r + kk, pl.ds(0, 16)]
                a1 = a1 + rows_v[r + kk, pl.ds(16, 16)]
            acc_v[e, pl.ds(0, 16)] = a0
            acc_v[e, pl.ds(16, 16)] = a1
            return carry

        lax.fori_loop(0, E_W, acc_edge, 0)
        pltpu.sync_copy(acc_v, out_hbm.at[pl.ds(base, E_W)])

    neg_side(neg_src, o_ns)
    neg_side(neg_dst, o_nd)


_sc_gather = functools.partial(
    pl.kernel,
    mesh=plsc.VectorSubcoreMesh(core_axis_name="c", subcore_axis_name="s"),
    out_type=[jax.ShapeDtypeStruct((B, D), jnp.float32)] * 4,
    scratch_types=[
        pltpu.VMEM((K, E_W), jnp.int32),
        pltpu.VMEM((K * E_W, D), jnp.float32),
        pltpu.VMEM((E_W, D), jnp.float32),
        pltpu.SemaphoreType.DMA,
    ],
    compiler_params=pltpu.CompilerParams(use_tc_tiling_on_sc=False),
)(_sc_gather_body)


def _repack_body(at_ref, eye_ref, out_ref):
    x = at_ref[...]
    q = RC // 4
    xc = jnp.concatenate(
        [x[:, 0:q], x[:, q:2 * q], x[:, 2 * q:3 * q], x[:, 3 * q:4 * q]],
        axis=0)
    out_ref[...] = jax.lax.dot_general(
        xc, eye_ref[...], (((0,), (0,)), ((), ())),
        preferred_element_type=jnp.float32)


_repack = pl.pallas_call(
    _repack_body,
    grid=(RG,),
    in_specs=[pl.BlockSpec((32, RC), lambda i: (0, i)),
              pl.BlockSpec((128, 128), lambda i: (0, 0))],
    out_specs=pl.BlockSpec((RC // 4, 128), lambda i: (i, 0)),
    out_shape=jax.ShapeDtypeStruct((PAD_ROWS // 4, 128), jnp.float32),
)


def _permute_idx(r):
    return (r & ~(RC - 1)) + ((r & (RC // 4 - 1)) << 2) + ((r & (RC - 1)) >> 11)


def _tc_loss_body(ps_ref, pd_ref, ns_ref, nd_ref, w_ref, out_ref):
    eps = 1e-8
    epsilon = 1e-7
    w = w_ref[...]

    def norm(x):
        rms = jnp.sqrt(jnp.sum(x * x, axis=-1, keepdims=True) * (1.0 / D))
        return x / (rms + eps) * w

    a = norm(ps_ref[...])
    b = norm(pd_ref[...])
    c = norm(ns_ref[...] * (1.0 / K))
    d = norm(nd_ref[...] * (1.0 / K))
    dpos = jnp.sum(a * b, axis=-1, keepdims=True)
    dneg = jnp.sum(c * d, axis=-1, keepdims=True)

    def log_sig(x):
        return jnp.log(1.0 / (1.0 + jnp.exp(-x)) + epsilon)

    def log_one_minus_sig(x):
        return jnp.log(1.0 - 1.0 / (1.0 + jnp.exp(-x)) + epsilon)

    one = log_sig(dpos * (1.0 / D)) + log_one_minus_sig(dneg * (1.0 / D))
    two = log_sig(dpos) + log_one_minus_sig(dneg)
    out_ref[0, 0] = -(jnp.sum(one) + jnp.sum(two)) * (1.0 / B)


_tc_loss = pl.pallas_call(
    _tc_loss_body,
    out_shape=jax.ShapeDtypeStruct((1, 1), jnp.float32),
    out_specs=pl.BlockSpec(memory_space=pltpu.SMEM),
)


def kernel(pos_edges, neg_edges, node_embeddings, rms_weight):
    pos_src = _permute_idx(pos_edges[0])
    pos_dst = _permute_idx(pos_edges[1])
    neg_src = _permute_idx(neg_edges[0]).reshape(NW, K, E_W)
    neg_dst = _permute_idx(neg_edges[1]).reshape(NW, K, E_W)
    table_lin = _repack(node_embeddings.T,
                        jnp.eye(128, dtype=jnp.float32)).reshape(PAD_ROWS, D)
    ps, pd, ns, nd = _sc_gather(table_lin, pos_src, pos_dst,
                                neg_src, neg_dst)
    loss = _tc_loss(ps, pd, ns, nd, rms_weight.reshape(1, D))
    return loss[0, 0]


# speedup vs baseline: 3.2802x; 1.2356x over previous
import functools

import jax
import jax.numpy as jnp
from jax import lax
from jax.experimental import pallas as pl
from jax.experimental.pallas import tpu as pltpu
from jax.experimental.pallas import tpu_sc as plsc

D = 32
B = 4096
K = 20
NUM_ROWS = 1000000
RC = 16384
RG = 62
PAD_ROWS = RG * RC
NC = 2
NS = 16
NW = NC * NS
E_W = B // NW


def _sc_gather_body(table, pos_src, pos_dst, neg_src, neg_dst,
                    o_ps, o_pd, o_ns, o_nd,
                    idx_v, rows_v, acc_v, sem):
    wid = lax.axis_index("s") * NC + lax.axis_index("c")
    base = wid * E_W

    def pos_side(idx_hbm, out_hbm):
        pltpu.sync_copy(idx_hbm.at[pl.ds(base, E_W)], idx_v.at[0])
        pltpu.async_copy(table.at[idx_v.at[0]],
                         rows_v.at[pl.ds(0, E_W)], sem).wait()
        pltpu.sync_copy(rows_v.at[pl.ds(0, E_W)],
                        out_hbm.at[pl.ds(base, E_W)])

    pos_side(pos_src, o_ps)
    pos_side(pos_dst, o_pd)

    def neg_side(idx_hbm, out_hbm):
        pltpu.sync_copy(idx_hbm.at[wid], idx_v)
        copies = [
            pltpu.async_copy(table.at[idx_v.at[c]],
                             rows_v.at[pl.ds(c * E_W, E_W)], sem)
            for c in range(K)
        ]
        for cp in copies:
            cp.wait()

        def acc_edge(e, carry):
            r = e * K
            a0 = rows_v[r, pl.ds(0, 16)]
            a1 = rows_v[r, pl.ds(16, 16)]
            for kk in range(1, K):
                a0 = a0 + rows_v[r + kk, pl.ds(0, 16)]
                a1 = a1 + rows_v[r + kk, pl.ds(16, 16)]
            acc_v[e, pl.ds(0, 16)] = a0
            acc_v[e, pl.ds(16, 16)] = a1
            return carry

        lax.fori_loop(0, E_W, acc_edge, 0)
        pltpu.sync_copy(acc_v, out_hbm.at[pl.ds(base, E_W)])

    neg_side(neg_src, o_ns)
    neg_side(neg_dst, o_nd)


_sc_gather = functools.partial(
    pl.kernel,
    mesh=plsc.VectorSubcoreMesh(core_axis_name="c", subcore_axis_name="s"),
    out_type=[jax.ShapeDtypeStruct((B, D), jnp.float32)] * 4,
    scratch_types=[
        pltpu.VMEM((K, E_W), jnp.int32),
        pltpu.VMEM((K * E_W, D), jnp.float32),
        pltpu.VMEM((E_W, D), jnp.float32),
        pltpu.SemaphoreType.DMA,
    ],
    compiler_params=pltpu.CompilerParams(use_tc_tiling_on_sc=False),
)(_sc_gather_body)


def _repack_body(at_ref, eye_ref, out_ref):
    x = at_ref[...]
    q = RC // 4
    xc = jnp.concatenate(
        [x[:, 0:q], x[:, q:2 * q], x[:, 2 * q:3 * q], x[:, 3 * q:4 * q]],
        axis=0)
    out_ref[...] = jax.lax.dot_general(
        xc, eye_ref[...], (((0,), (0,)), ((), ())),
        preferred_element_type=jnp.float32)


_repack = pl.pallas_call(
    _repack_body,
    grid=(RG,),
    in_specs=[pl.BlockSpec((32, RC), lambda i: (0, i)),
              pl.BlockSpec((128, 128), lambda i: (0, 0))],
    out_specs=pl.BlockSpec((RC // 4, 128), lambda i: (i, 0)),
    out_shape=jax.ShapeDtypeStruct((PAD_ROWS // 4, 128), jnp.float32),
)


_QSHIFT = (RC // 4).bit_length() - 1


def _permute_idx(r):
    return ((r & ~(RC - 1)) + ((r & (RC // 4 - 1)) << 2)
            + ((r & (RC - 1)) >> _QSHIFT))


def _tc_loss_body(ps_ref, pd_ref, ns_ref, nd_ref, w_ref, out_ref):
    eps = 1e-8
    epsilon = 1e-7
    w = w_ref[...]

    def norm(x):
        rms = jnp.sqrt(jnp.sum(x * x, axis=-1, keepdims=True) * (1.0 / D))
        return x / (rms + eps) * w

    a = norm(ps_ref[...])
    b = norm(pd_ref[...])
    c = norm(ns_ref[...] * (1.0 / K))
    d = norm(nd_ref[...] * (1.0 / K))
    dpos = jnp.sum(a * b, axis=-1, keepdims=True)
    dneg = jnp.sum(c * d, axis=-1, keepdims=True)

    def log_sig(x):
        return jnp.log(1.0 / (1.0 + jnp.exp(-x)) + epsilon)

    def log_one_minus_sig(x):
        return jnp.log(1.0 - 1.0 / (1.0 + jnp.exp(-x)) + epsilon)

    one = log_sig(dpos * (1.0 / D)) + log_one_minus_sig(dneg * (1.0 / D))
    two = log_sig(dpos) + log_one_minus_sig(dneg)
    out_ref[0, 0] = -(jnp.sum(one) + jnp.sum(two)) * (1.0 / B)


_tc_loss = pl.pallas_call(
    _tc_loss_body,
    out_shape=jax.ShapeDtypeStruct((1, 1), jnp.float32),
    out_specs=pl.BlockSpec(memory_space=pltpu.SMEM),
)


def kernel(pos_edges, neg_edges, node_embeddings, rms_weight):
    pos_src = _permute_idx(pos_edges[0])
    pos_dst = _permute_idx(pos_edges[1])
    neg_src = _permute_idx(neg_edges[0]).reshape(NW, K, E_W)
    neg_dst = _permute_idx(neg_edges[1]).reshape(NW, K, E_W)
    table_lin = _repack(node_embeddings.T,
                        jnp.eye(128, dtype=jnp.float32)).reshape(PAD_ROWS, D)
    ps, pd, ns, nd = _sc_gather(table_lin, pos_src, pos_dst,
                                neg_src, neg_dst)
    loss = _tc_loss(ps, pd, ns, nd, rms_weight.reshape(1, D))
    return loss[0, 0]


# speedup vs baseline: 3.6048x; 1.0990x over previous
import functools

import jax
import jax.numpy as jnp
from jax import lax
from jax.experimental import pallas as pl
from jax.experimental.pallas import tpu as pltpu
from jax.experimental.pallas import tpu_sc as plsc

D = 32
B = 4096
K = 20
NUM_ROWS = 1000000
RC = 32768
RG = 31
PAD_ROWS = RG * RC
NC = 2
NS = 16
NW = NC * NS
E_W = B // NW


def _sc_gather_body(table, pos_src, pos_dst, neg_src, neg_dst,
                    o_ps, o_pd, o_ns, o_nd,
                    idx_v, rows_v, acc_v, sem):
    wid = lax.axis_index("s") * NC + lax.axis_index("c")
    base = wid * E_W

    def pos_side(idx_hbm, out_hbm):
        pltpu.sync_copy(idx_hbm.at[pl.ds(base, E_W)], idx_v.at[0])
        pltpu.async_copy(table.at[idx_v.at[0]],
                         rows_v.at[pl.ds(0, E_W)], sem).wait()
        pltpu.sync_copy(rows_v.at[pl.ds(0, E_W)],
                        out_hbm.at[pl.ds(base, E_W)])

    pos_side(pos_src, o_ps)
    pos_side(pos_dst, o_pd)

    def neg_side(idx_hbm, out_hbm):
        pltpu.sync_copy(idx_hbm.at[wid], idx_v)
        copies = [
            pltpu.async_copy(table.at[idx_v.at[c]],
                             rows_v.at[pl.ds(c * E_W, E_W)], sem)
            for c in range(K)
        ]
        for cp in copies:
            cp.wait()

        def acc_edge(e, carry):
            r = e * K
            a0 = rows_v[r, pl.ds(0, 16)]
            a1 = rows_v[r, pl.ds(16, 16)]
            for kk in range(1, K):
                a0 = a0 + rows_v[r + kk, pl.ds(0, 16)]
                a1 = a1 + rows_v[r + kk, pl.ds(16, 16)]
            acc_v[e, pl.ds(0, 16)] = a0
            acc_v[e, pl.ds(16, 16)] = a1
            return carry

        lax.fori_loop(0, E_W, acc_edge, 0)
        pltpu.sync_copy(acc_v, out_hbm.at[pl.ds(base, E_W)])

    neg_side(neg_src, o_ns)
    neg_side(neg_dst, o_nd)


_sc_gather = functools.partial(
    pl.kernel,
    mesh=plsc.VectorSubcoreMesh(core_axis_name="c", subcore_axis_name="s"),
    out_type=[jax.ShapeDtypeStruct((B, D), jnp.float32)] * 4,
    scratch_types=[
        pltpu.VMEM((K, E_W), jnp.int32),
        pltpu.VMEM((K * E_W, D), jnp.float32),
        pltpu.VMEM((E_W, D), jnp.float32),
        pltpu.SemaphoreType.DMA,
    ],
    compiler_params=pltpu.CompilerParams(use_tc_tiling_on_sc=False),
)(_sc_gather_body)


def _repack_body(at_ref, eye_ref, out_ref):
    x = at_ref[...]
    q = RC // 4
    xc = jnp.concatenate(
        [x[:, 0:q], x[:, q:2 * q], x[:, 2 * q:3 * q], x[:, 3 * q:4 * q]],
        axis=0)
    out_ref[...] = jax.lax.dot_general(
        xc, eye_ref[...], (((0,), (0,)), ((), ())),
        preferred_element_type=jnp.float32)


_repack = pl.pallas_call(
    _repack_body,
    grid=(RG,),
    in_specs=[pl.BlockSpec((32, RC), lambda i: (0, i)),
              pl.BlockSpec((128, 128), lambda i: (0, 0))],
    out_specs=pl.BlockSpec((RC // 4, 128), lambda i: (i, 0)),
    out_shape=jax.ShapeDtypeStruct((PAD_ROWS // 4, 128), jnp.float32),
)


_QSHIFT = (RC // 4).bit_length() - 1


def _permute_idx(r):
    return ((r & ~(RC - 1)) + ((r & (RC // 4 - 1)) << 2)
            + ((r & (RC - 1)) >> _QSHIFT))


def _tc_loss_body(ps_ref, pd_ref, ns_ref, nd_ref, w_ref, out_ref):
    eps = 1e-8
    epsilon = 1e-7
    w = w_ref[...]

    def norm(x):
        rms = jnp.sqrt(jnp.sum(x * x, axis=-1, keepdims=True) * (1.0 / D))
        return x / (rms + eps) * w

    a = norm(ps_ref[...])
    b = norm(pd_ref[...])
    c = norm(ns_ref[...] * (1.0 / K))
    d = norm(nd_ref[...] * (1.0 / K))
    dpos = jnp.sum(a * b, axis=-1, keepdims=True)
    dneg = jnp.sum(c * d, axis=-1, keepdims=True)

    def log_sig(x):
        return jnp.log(1.0 / (1.0 + jnp.exp(-x)) + epsilon)

    def log_one_minus_sig(x):
        return jnp.log(1.0 - 1.0 / (1.0 + jnp.exp(-x)) + epsilon)

    one = log_sig(dpos * (1.0 / D)) + log_one_minus_sig(dneg * (1.0 / D))
    two = log_sig(dpos) + log_one_minus_sig(dneg)
    out_ref[0, 0] = -(jnp.sum(one) + jnp.sum(two)) * (1.0 / B)


_tc_loss = pl.pallas_call(
    _tc_loss_body,
    out_shape=jax.ShapeDtypeStruct((1, 1), jnp.float32),
    out_specs=pl.BlockSpec(memory_space=pltpu.SMEM),
)


def kernel(pos_edges, neg_edges, node_embeddings, rms_weight):
    pos_src = _permute_idx(pos_edges[0])
    pos_dst = _permute_idx(pos_edges[1])
    neg_src = _permute_idx(neg_edges[0]).reshape(NW, K, E_W)
    neg_dst = _permute_idx(neg_edges[1]).reshape(NW, K, E_W)
    table_lin = _repack(node_embeddings.T,
                        jnp.eye(128, dtype=jnp.float32)).reshape(PAD_ROWS, D)
    ps, pd, ns, nd = _sc_gather(table_lin, pos_src, pos_dst,
                                neg_src, neg_dst)
    loss = _tc_loss(ps, pd, ns, nd, rms_weight.reshape(1, D))
    return loss[0, 0]


# speedup vs baseline: 3.6292x; 1.0068x over previous
import functools

import jax
import jax.numpy as jnp
from jax import lax
from jax.experimental import pallas as pl
from jax.experimental.pallas import tpu as pltpu
from jax.experimental.pallas import tpu_sc as plsc

D = 32
B = 4096
K = 20
NUM_ROWS = 1000000
RC = 65536
RG = 16
PAD_ROWS = RG * RC
NC = 2
NS = 16
NW = NC * NS
E_W = B // NW


def _sc_gather_body(table, pos_src, pos_dst, neg_src, neg_dst,
                    o_ps, o_pd, o_ns, o_nd,
                    idx_v, rows_v, acc_v, sem):
    wid = lax.axis_index("s") * NC + lax.axis_index("c")
    base = wid * E_W

    def pos_side(idx_hbm, out_hbm):
        pltpu.sync_copy(idx_hbm.at[pl.ds(base, E_W)], idx_v.at[0])
        pltpu.async_copy(table.at[idx_v.at[0]],
                         rows_v.at[pl.ds(0, E_W)], sem).wait()
        pltpu.sync_copy(rows_v.at[pl.ds(0, E_W)],
                        out_hbm.at[pl.ds(base, E_W)])

    pos_side(pos_src, o_ps)
    pos_side(pos_dst, o_pd)

    def neg_side(idx_hbm, out_hbm):
        pltpu.sync_copy(idx_hbm.at[wid], idx_v)
        copies = [
            pltpu.async_copy(table.at[idx_v.at[c]],
                             rows_v.at[pl.ds(c * E_W, E_W)], sem)
            for c in range(K)
        ]
        for cp in copies:
            cp.wait()

        def acc_edge(e, carry):
            r = e * K
            a0 = rows_v[r, pl.ds(0, 16)]
            a1 = rows_v[r, pl.ds(16, 16)]
            for kk in range(1, K):
                a0 = a0 + rows_v[r + kk, pl.ds(0, 16)]
                a1 = a1 + rows_v[r + kk, pl.ds(16, 16)]
            acc_v[e, pl.ds(0, 16)] = a0
            acc_v[e, pl.ds(16, 16)] = a1
            return carry

        lax.fori_loop(0, E_W, acc_edge, 0)
        pltpu.sync_copy(acc_v, out_hbm.at[pl.ds(base, E_W)])

    neg_side(neg_src, o_ns)
    neg_side(neg_dst, o_nd)


_sc_gather = functools.partial(
    pl.kernel,
    mesh=plsc.VectorSubcoreMesh(core_axis_name="c", subcore_axis_name="s"),
    out_type=[jax.ShapeDtypeStruct((B, D), jnp.float32)] * 4,
    scratch_types=[
        pltpu.VMEM((K, E_W), jnp.int32),
        pltpu.VMEM((K * E_W, D), jnp.float32),
        pltpu.VMEM((E_W, D), jnp.float32),
        pltpu.SemaphoreType.DMA,
    ],
    compiler_params=pltpu.CompilerParams(use_tc_tiling_on_sc=False),
)(_sc_gather_body)


def _repack_body(at_ref, eye_ref, out_ref):
    x = at_ref[...]
    q = RC // 4
    xc = jnp.concatenate(
        [x[:, 0:q], x[:, q:2 * q], x[:, 2 * q:3 * q], x[:, 3 * q:4 * q]],
        axis=0)
    out_ref[...] = jax.lax.dot_general(
        xc, eye_ref[...], (((0,), (0,)), ((), ())),
        preferred_element_type=jnp.float32)


_repack = pl.pallas_call(
    _repack_body,
    grid=(RG,),
    in_specs=[pl.BlockSpec((32, RC), lambda i: (0, i)),
              pl.BlockSpec((128, 128), lambda i: (0, 0))],
    out_specs=pl.BlockSpec((RC // 4, 128), lambda i: (i, 0)),
    out_shape=jax.ShapeDtypeStruct((PAD_ROWS // 4, 128), jnp.float32),
)


_QSHIFT = (RC // 4).bit_length() - 1


def _permute_idx(r):
    return ((r & ~(RC - 1)) + ((r & (RC // 4 - 1)) << 2)
            + ((r & (RC - 1)) >> _QSHIFT))


def _tc_loss_body(ps_ref, pd_ref, ns_ref, nd_ref, w_ref, out_ref):
    eps = 1e-8
    epsilon = 1e-7
    w = w_ref[...]

    def norm(x):
        rms = jnp.sqrt(jnp.sum(x * x, axis=-1, keepdims=True) * (1.0 / D))
        return x / (rms + eps) * w

    a = norm(ps_ref[...])
    b = norm(pd_ref[...])
    c = norm(ns_ref[...] * (1.0 / K))
    d = norm(nd_ref[...] * (1.0 / K))
    dpos = jnp.sum(a * b, axis=-1, keepdims=True)
    dneg = jnp.sum(c * d, axis=-1, keepdims=True)

    def log_sig(x):
        return jnp.log(1.0 / (1.0 + jnp.exp(-x)) + epsilon)

    def log_one_minus_sig(x):
        return jnp.log(1.0 - 1.0 / (1.0 + jnp.exp(-x)) + epsilon)

    one = log_sig(dpos * (1.0 / D)) + log_one_minus_sig(dneg * (1.0 / D))
    two = log_sig(dpos) + log_one_minus_sig(dneg)
    out_ref[0, 0] = -(jnp.sum(one) + jnp.sum(two)) * (1.0 / B)


_tc_loss = pl.pallas_call(
    _tc_loss_body,
    out_shape=jax.ShapeDtypeStruct((1, 1), jnp.float32),
    out_specs=pl.BlockSpec(memory_space=pltpu.SMEM),
)


def kernel(pos_edges, neg_edges, node_embeddings, rms_weight):
    pos_src = _permute_idx(pos_edges[0])
    pos_dst = _permute_idx(pos_edges[1])
    neg_src = _permute_idx(neg_edges[0]).reshape(NW, K, E_W)
    neg_dst = _permute_idx(neg_edges[1]).reshape(NW, K, E_W)
    table_lin = _repack(node_embeddings.T,
                        jnp.eye(128, dtype=jnp.float32)).reshape(PAD_ROWS, D)
    ps, pd, ns, nd = _sc_gather(table_lin, pos_src, pos_dst,
                                neg_src, neg_dst)
    loss = _tc_loss(ps, pd, ns, nd, rms_weight.reshape(1, D))
    return loss[0, 0]


# speedup vs baseline: 3.9323x; 1.0835x over previous
import functools

import jax
import jax.numpy as jnp
from jax import lax
from jax.experimental import pallas as pl
from jax.experimental.pallas import tpu as pltpu
from jax.experimental.pallas import tpu_sc as plsc

D = 32
B = 4096
K = 20
NUM_ROWS = 1000000
RC = 65536
RG = 16
PAD_ROWS = RG * RC
NC = 2
NS = 16
NW = NC * NS
E_W = B // NW


def _sc_gather_body(table, pos_src, pos_dst, neg_src, neg_dst,
                    o_ps, o_pd, o_ns, o_nd,
                    idx_v, rows_v, acc_v, sem):
    wid = lax.axis_index("s") * NC + lax.axis_index("c")
    base = wid * E_W

    mask_hi = jnp.int32(-65536)

    def unpack(v):
        h = plsc.bitcast(v & mask_hi, jnp.float32)
        l = plsc.bitcast(v << 16, jnp.float32)
        return h, l

    def pos_side(idx_hbm, out_hbm):
        pltpu.sync_copy(idx_hbm.at[pl.ds(base, E_W)], idx_v.at[0])
        pltpu.async_copy(table.at[idx_v.at[0]],
                         rows_v.at[pl.ds(0, E_W)], sem).wait()

        def unpack_row(e, carry):
            h, l = unpack(rows_v[e, :])
            acc_v[e, pl.ds(0, 16)] = h
            acc_v[e, pl.ds(16, 16)] = l
            return carry

        lax.fori_loop(0, E_W, unpack_row, 0)
        pltpu.sync_copy(acc_v, out_hbm.at[pl.ds(base, E_W)])

    pos_side(pos_src, o_ps)
    pos_side(pos_dst, o_pd)

    def neg_side(idx_hbm, out_hbm):
        pltpu.sync_copy(idx_hbm.at[wid], idx_v)
        copies = [
            pltpu.async_copy(table.at[idx_v.at[c]],
                             rows_v.at[pl.ds(c * E_W, E_W)], sem)
            for c in range(K)
        ]
        for cp in copies:
            cp.wait()

        def acc_edge(e, carry):
            r = e * K
            a0, a1 = unpack(rows_v[r, :])
            for kk in range(1, K):
                h, l = unpack(rows_v[r + kk, :])
                a0 = a0 + h
                a1 = a1 + l
            acc_v[e, pl.ds(0, 16)] = a0
            acc_v[e, pl.ds(16, 16)] = a1
            return carry

        lax.fori_loop(0, E_W, acc_edge, 0)
        pltpu.sync_copy(acc_v, out_hbm.at[pl.ds(base, E_W)])

    neg_side(neg_src, o_ns)
    neg_side(neg_dst, o_nd)


_sc_gather = functools.partial(
    pl.kernel,
    mesh=plsc.VectorSubcoreMesh(core_axis_name="c", subcore_axis_name="s"),
    out_type=[jax.ShapeDtypeStruct((B, D), jnp.float32)] * 4,
    scratch_types=[
        pltpu.VMEM((K, E_W), jnp.int32),
        pltpu.VMEM((K * E_W, 16), jnp.int32),
        pltpu.VMEM((E_W, D), jnp.float32),
        pltpu.SemaphoreType.DMA,
    ],
    compiler_params=pltpu.CompilerParams(use_tc_tiling_on_sc=False,
                                         needs_layout_passes=False),
)(_sc_gather_body)


def _repack_body(at_ref, ehi_ref, elo_ref, out_ref):
    i = pl.program_id(0)
    x = at_ref[...]
    cols = i * RC + jax.lax.broadcasted_iota(jnp.int32, (32, RC), 1)
    x = jnp.where(cols < NUM_ROWS, x, 0.0)
    s = RC // 8
    xc = jnp.concatenate([x[:, j * s:(j + 1) * s] for j in range(8)],
                         axis=0)
    dims = (((0,), (0,)), ((), ()))
    hi = jax.lax.dot_general(xc, ehi_ref[...], dims,
                             preferred_element_type=jnp.float32)
    lo = jax.lax.dot_general(xc, elo_ref[...], dims,
                             preferred_element_type=jnp.float32)
    hi_i = jax.lax.bitcast_convert_type(hi, jnp.int32)
    lo_i = jax.lax.bitcast_convert_type(lo, jnp.int32)
    hi_b = (hi_i + 32768) & jnp.int32(-65536)
    lo_b = jax.lax.shift_right_logical(lo_i + 32768, 16)
    out_ref[...] = hi_b | lo_b


_repack = pl.pallas_call(
    _repack_body,
    grid=(RG,),
    in_specs=[pl.BlockSpec((32, RC), lambda i: (0, i)),
              pl.BlockSpec((256, 128), lambda i: (0, 0)),
              pl.BlockSpec((256, 128), lambda i: (0, 0))],
    out_specs=pl.BlockSpec((RC // 8, 128), lambda i: (i, 0)),
    out_shape=jax.ShapeDtypeStruct((PAD_ROWS // 8, 128), jnp.int32),
)


_QSHIFT = (RC // 8).bit_length() - 1


def _permute_idx(r):
    return ((r & ~(RC - 1)) + ((r & (RC // 8 - 1)) << 3)
            + ((r & (RC - 1)) >> _QSHIFT))


def _selection_mats():
    p = jnp.arange(256, dtype=jnp.int32)[:, None]
    c = jnp.arange(128, dtype=jnp.int32)[None, :]
    base = (c // 16) * 32 + (c % 16) * 2
    ehi = (p == base).astype(jnp.float32)
    elo = (p == base + 1).astype(jnp.float32)
    return ehi, elo


def _tc_loss_body(ps_ref, pd_ref, ns_ref, nd_ref, w_ref, out_ref):
    eps = 1e-8
    epsilon = 1e-7
    w = w_ref[...]

    def norm(x):
        rms = jnp.sqrt(jnp.sum(x * x, axis=-1, keepdims=True) * (1.0 / D))
        return x / (rms + eps) * w

    a = norm(ps_ref[...])
    b = norm(pd_ref[...])
    c = norm(ns_ref[...] * (1.0 / K))
    d = norm(nd_ref[...] * (1.0 / K))
    dpos = jnp.sum(a * b, axis=-1, keepdims=True)
    dneg = jnp.sum(c * d, axis=-1, keepdims=True)

    def log_sig(x):
        return jnp.log(1.0 / (1.0 + jnp.exp(-x)) + epsilon)

    def log_one_minus_sig(x):
        return jnp.log(1.0 - 1.0 / (1.0 + jnp.exp(-x)) + epsilon)

    one = log_sig(dpos * (1.0 / D)) + log_one_minus_sig(dneg * (1.0 / D))
    two = log_sig(dpos) + log_one_minus_sig(dneg)
    out_ref[0, 0] = -(jnp.sum(one) + jnp.sum(two)) * (1.0 / B)


_tc_loss = pl.pallas_call(
    _tc_loss_body,
    out_shape=jax.ShapeDtypeStruct((1, 1), jnp.float32),
    out_specs=pl.BlockSpec(memory_space=pltpu.SMEM),
)


def kernel(pos_edges, neg_edges, node_embeddings, rms_weight):
    pos_src = _permute_idx(pos_edges[0])
    pos_dst = _permute_idx(pos_edges[1])
    neg_src = _permute_idx(neg_edges[0]).reshape(NW, K, E_W)
    neg_dst = _permute_idx(neg_edges[1]).reshape(NW, K, E_W)
    ehi, elo = _selection_mats()
    table_pk = _repack(node_embeddings.T, ehi, elo).reshape(PAD_ROWS, 16)
    ps, pd, ns, nd = _sc_gather(table_pk, pos_src, pos_dst,
                                neg_src, neg_dst)
    w_perm = jnp.concatenate([rms_weight[0::2], rms_weight[1::2]])
    loss = _tc_loss(ps, pd, ns, nd, w_perm.reshape(1, D))
    return loss[0, 0]


# speedup vs baseline: 4.0409x; 1.0276x over previous
import functools

import jax
import jax.numpy as jnp
from jax import lax
from jax.experimental import pallas as pl
from jax.experimental.pallas import tpu as pltpu
from jax.experimental.pallas import tpu_sc as plsc

D = 32
B = 4096
K = 20
NUM_ROWS = 1000000
RC = 131072
RG = 8
PAD_ROWS = RG * RC
NC = 2
NS = 16
NW = NC * NS
E_W = B // NW


def _sc_gather_body(table, pos_src, pos_dst, neg_src, neg_dst,
                    o_ps, o_pd, o_ns, o_nd,
                    idx_v, rows_v, acc_v, sem):
    wid = lax.axis_index("s") * NC + lax.axis_index("c")
    base = wid * E_W

    mask_hi = jnp.int32(-65536)

    def unpack(v):
        h = plsc.bitcast(v & mask_hi, jnp.float32)
        l = plsc.bitcast(v << 16, jnp.float32)
        return h, l

    def pos_side(idx_hbm, out_hbm):
        pltpu.sync_copy(idx_hbm.at[pl.ds(base, E_W)], idx_v.at[0])
        pltpu.async_copy(table.at[idx_v.at[0]],
                         rows_v.at[pl.ds(0, E_W)], sem).wait()

        def unpack_row(e, carry):
            h, l = unpack(rows_v[e, :])
            acc_v[e, pl.ds(0, 16)] = h
            acc_v[e, pl.ds(16, 16)] = l
            return carry

        lax.fori_loop(0, E_W, unpack_row, 0)
        pltpu.sync_copy(acc_v, out_hbm.at[pl.ds(base, E_W)])

    pos_side(pos_src, o_ps)
    pos_side(pos_dst, o_pd)

    def neg_side(idx_hbm, out_hbm):
        pltpu.sync_copy(idx_hbm.at[wid], idx_v)
        copies = [
            pltpu.async_copy(table.at[idx_v.at[c]],
                             rows_v.at[pl.ds(c * E_W, E_W)], sem)
            for c in range(K)
        ]
        for cp in copies:
            cp.wait()

        def acc_edge(e, carry):
            r = e * K
            a0, a1 = unpack(rows_v[r, :])
            for kk in range(1, K):
                h, l = unpack(rows_v[r + kk, :])
                a0 = a0 + h
                a1 = a1 + l
            acc_v[e, pl.ds(0, 16)] = a0
            acc_v[e, pl.ds(16, 16)] = a1
            return carry

        lax.fori_loop(0, E_W, acc_edge, 0)
        pltpu.sync_copy(acc_v, out_hbm.at[pl.ds(base, E_W)])

    neg_side(neg_src, o_ns)
    neg_side(neg_dst, o_nd)


_sc_gather = functools.partial(
    pl.kernel,
    mesh=plsc.VectorSubcoreMesh(core_axis_name="c", subcore_axis_name="s"),
    out_type=[jax.ShapeDtypeStruct((B, D), jnp.float32)] * 4,
    scratch_types=[
        pltpu.VMEM((K, E_W), jnp.int32),
        pltpu.VMEM((K * E_W, 16), jnp.int32),
        pltpu.VMEM((E_W, D), jnp.float32),
        pltpu.SemaphoreType.DMA,
    ],
    compiler_params=pltpu.CompilerParams(use_tc_tiling_on_sc=False,
                                         needs_layout_passes=False),
)(_sc_gather_body)


def _repack_body(at_ref, ehi_ref, elo_ref, out_ref):
    i = pl.program_id(0)
    x = at_ref[...]
    cols = i * RC + jax.lax.broadcasted_iota(jnp.int32, (32, RC), 1)
    x = jnp.where(cols < NUM_ROWS, x, 0.0)
    s = RC // 8
    xc = jnp.concatenate([x[:, j * s:(j + 1) * s] for j in range(8)],
                         axis=0)
    dims = (((0,), (0,)), ((), ()))
    hi = jax.lax.dot_general(xc, ehi_ref[...], dims,
                             preferred_element_type=jnp.float32)
    lo = jax.lax.dot_general(xc, elo_ref[...], dims,
                             preferred_element_type=jnp.float32)
    hi_i = jax.lax.bitcast_convert_type(hi, jnp.int32)
    lo_i = jax.lax.bitcast_convert_type(lo, jnp.int32)
    hi_b = (hi_i + 32768) & jnp.int32(-65536)
    lo_b = jax.lax.shift_right_logical(lo_i + 32768, 16)
    out_ref[...] = hi_b | lo_b


_repack = pl.pallas_call(
    _repack_body,
    grid=(RG,),
    in_specs=[pl.BlockSpec((32, RC), lambda i: (0, i)),
              pl.BlockSpec((256, 128), lambda i: (0, 0)),
              pl.BlockSpec((256, 128), lambda i: (0, 0))],
    out_specs=pl.BlockSpec((RC // 8, 128), lambda i: (i, 0)),
    out_shape=jax.ShapeDtypeStruct((PAD_ROWS // 8, 128), jnp.int32),
)


_QSHIFT = (RC // 8).bit_length() - 1


def _permute_idx(r):
    return ((r & ~(RC - 1)) + ((r & (RC // 8 - 1)) << 3)
            + ((r & (RC - 1)) >> _QSHIFT))


def _selection_mats():
    p = jnp.arange(256, dtype=jnp.int32)[:, None]
    c = jnp.arange(128, dtype=jnp.int32)[None, :]
    base = (c // 16) * 32 + (c % 16) * 2
    ehi = (p == base).astype(jnp.float32)
    elo = (p == base + 1).astype(jnp.float32)
    return ehi, elo


def _tc_loss_body(ps_ref, pd_ref, ns_ref, nd_ref, w_ref, out_ref):
    eps = 1e-8
    epsilon = 1e-7
    w = w_ref[...]

    def norm(x):
        rms = jnp.sqrt(jnp.sum(x * x, axis=-1, keepdims=True) * (1.0 / D))
        return x / (rms + eps) * w

    a = norm(ps_ref[...])
    b = norm(pd_ref[...])
    c = norm(ns_ref[...] * (1.0 / K))
    d = norm(nd_ref[...] * (1.0 / K))
    dpos = jnp.sum(a * b, axis=-1, keepdims=True)
    dneg = jnp.sum(c * d, axis=-1, keepdims=True)

    def log_sig(x):
        return jnp.log(1.0 / (1.0 + jnp.exp(-x)) + epsilon)

    def log_one_minus_sig(x):
        return jnp.log(1.0 - 1.0 / (1.0 + jnp.exp(-x)) + epsilon)

    one = log_sig(dpos * (1.0 / D)) + log_one_minus_sig(dneg * (1.0 / D))
    two = log_sig(dpos) + log_one_minus_sig(dneg)
    out_ref[0, 0] = -(jnp.sum(one) + jnp.sum(two)) * (1.0 / B)


_tc_loss = pl.pallas_call(
    _tc_loss_body,
    out_shape=jax.ShapeDtypeStruct((1, 1), jnp.float32),
    out_specs=pl.BlockSpec(memory_space=pltpu.SMEM),
)


def kernel(pos_edges, neg_edges, node_embeddings, rms_weight):
    pos_src = _permute_idx(pos_edges[0])
    pos_dst = _permute_idx(pos_edges[1])
    neg_src = _permute_idx(neg_edges[0]).reshape(NW, K, E_W)
    neg_dst = _permute_idx(neg_edges[1]).reshape(NW, K, E_W)
    ehi, elo = _selection_mats()
    table_pk = _repack(node_embeddings.T, ehi, elo).reshape(PAD_ROWS, 16)
    ps, pd, ns, nd = _sc_gather(table_pk, pos_src, pos_dst,
                                neg_src, neg_dst)
    w_perm = jnp.concatenate([rms_weight[0::2], rms_weight[1::2]])
    loss = _tc_loss(ps, pd, ns, nd, w_perm.reshape(1, D))
    return loss[0, 0]
